# baseline (speedup 1.0000x reference)
import functools
import math

import jax
import jax.numpy as jnp
from jax import lax
from jax.experimental import pallas as pl
from jax.experimental.pallas import tpu as pltpu
from jax.experimental.pallas import tpu_sc as plsc

D_MODEL = 64
LANES = 16
NUM_CORES = 2
NUM_SUBCORES = 16
NUM_WORKERS = NUM_CORES * NUM_SUBCORES
GROUP = 128
CHUNK = 128


def _build_gather(seq, tokens):
    t_per_w = tokens // NUM_WORKERS
    halves = t_per_w // CHUNK
    n = seq * halves
    mesh = plsc.VectorSubcoreMesh(
        core_axis_name="c", subcore_axis_name="s",
        num_cores=NUM_CORES, num_subcores=NUM_SUBCORES)

    @functools.partial(
        pl.kernel,
        out_type=jax.ShapeDtypeStruct((seq * tokens, D_MODEL), jnp.float32),
        mesh=mesh,
        scratch_types=[
            pltpu.VMEM((seq * t_per_w,), jnp.int32),
            pltpu.VMEM((2, CHUNK, 128), jnp.float32),
            pltpu.VMEM((2, CHUNK, D_MODEL), jnp.float32),
            [pltpu.SemaphoreType.DMA] * 2,
            [pltpu.SemaphoreType.DMA] * 2,
        ],
        compiler_params=pltpu.CompilerParams(needs_layout_passes=False),
    )
    def emb_kernel(tbl_hbm, xw_hbm, out_hbm, xv, rows, outb, gsem, osem):
        wid = lax.axis_index("s") * NUM_CORES + lax.axis_index("c")
        t0 = wid * t_per_w

        def pos(c):
            s = c // halves
            tch = t0 + (c % halves) * CHUNK
            return s, tch, pl.multiple_of(s * tokens + tch, CHUNK)

        def gather_start(c, slot):
            base = pl.multiple_of(c * CHUNK, CHUNK)
            pltpu.async_copy(
                tbl_hbm.at[xv.at[pl.ds(base, CHUNK)]],
                rows.at[slot], gsem[slot])

        def gather_wait(c, slot):
            base = pl.multiple_of(c * CHUNK, CHUNK)
            pltpu.make_async_copy(
                tbl_hbm.at[xv.at[pl.ds(base, CHUNK)]],
                rows.at[slot], gsem[slot]).wait()

        def store_start(c, slot):
            _, _, row0 = pos(c)
            pltpu.async_copy(
                outb.at[slot], out_hbm.at[pl.ds(row0, CHUNK)], osem[slot])

        def store_wait(c, slot):
            _, _, row0 = pos(c)
            pltpu.make_async_copy(
                outb.at[slot], out_hbm.at[pl.ds(row0, CHUNK)], osem[slot]).wait()

        def compute(slot):
            @pl.loop(0, CHUNK, unroll=4)
            def _r(r):
                for dd in range(D_MODEL // LANES):
                    sl = pl.ds(dd * LANES, LANES)
                    outb[slot, r, sl] = rows[slot, r, sl] * 8.0

        pltpu.sync_copy(xw_hbm.at[wid], xv)
        gather_start(0, 0)

        @pl.loop(0, n, step=2)
        def _chunks(c0):
            for b in range(2):
                c = c0 + b
                slot = b
                nslot = 1 - b

                @pl.when(c + 1 < n)
                def _():
                    gather_start(c + 1, nslot)

                gather_wait(c, slot)

                @pl.when(c >= 2)
                def _():
                    store_wait(c - 2, slot)

                compute(slot)
                store_start(c, slot)

        store_wait(n - 2, 0)
        store_wait(n - 1, 1)

    return emb_kernel


def _tc_transpose(seq, tokens):
    tb = 1024
    grid = (seq, tokens // tb)

    def body(in_ref, out_ref):
        r = lax.broadcasted_iota(jnp.int32, (D_MODEL, D_MODEL), 0)
        c = lax.broadcasted_iota(jnp.int32, (D_MODEL, D_MODEL), 1)
        eye = (r == c).astype(jnp.float32)
        out_ref[...] = lax.dot_general(
            eye, in_ref[...], (((1,), (1,)), ((), ())),
            preferred_element_type=jnp.float32,
            precision=lax.Precision.HIGHEST)[None]

    return pl.pallas_call(
        body,
        grid=grid,
        in_specs=[
            pl.BlockSpec((tb, D_MODEL),
                         lambda s, t: (s * (tokens // tb) + t, 0)),
        ],
        out_specs=pl.BlockSpec((1, D_MODEL, tb), lambda s, t: (s, 0, t)),
        out_shape=jax.ShapeDtypeStruct((seq, D_MODEL, tokens), jnp.float32),
    )


def kernel(x, table):
    s0, s1 = x.shape
    vocab, d = table.shape
    t_per_w = s0 // NUM_WORKERS
    xt = x.astype(jnp.int32).T
    xw = (xt.reshape(s1, NUM_WORKERS, t_per_w)
          .transpose(1, 0, 2).reshape(NUM_WORKERS, s1 * t_per_w))
    tbl = jnp.pad(table, ((0, 0), (0, d)))
    emb2 = _build_gather(s1, s0)(tbl, xw)
    outt = _tc_transpose(s1, s0)(emb2)
    return outt.transpose(2, 0, 1)


# speedup vs baseline: 1.3693x; 1.3693x over previous
import functools
import math

import jax
import jax.numpy as jnp
from jax import lax
from jax.experimental import pallas as pl
from jax.experimental.pallas import tpu as pltpu
from jax.experimental.pallas import tpu_sc as plsc

D_MODEL = 64
LANES = 16
NUM_CORES = 2
NUM_SUBCORES = 16
NUM_WORKERS = NUM_CORES * NUM_SUBCORES
GROUP = 128
NBUF = 4
PDIST = 2


def _build(num_groups):
    groups_per_worker = num_groups // NUM_WORKERS
    mesh = plsc.VectorSubcoreMesh(
        core_axis_name="c", subcore_axis_name="s",
        num_cores=NUM_CORES, num_subcores=NUM_SUBCORES)

    @functools.partial(
        pl.kernel,
        out_type=jax.ShapeDtypeStruct((num_groups, GROUP, D_MODEL), jnp.float32),
        mesh=mesh,
        scratch_types=[
            pltpu.VMEM((groups_per_worker, GROUP), jnp.int32),
            pltpu.VMEM((NBUF, GROUP, D_MODEL), jnp.float32),
            [pltpu.SemaphoreType.DMA] * NBUF,
            [pltpu.SemaphoreType.DMA] * NBUF,
        ],
        compiler_params=pltpu.CompilerParams(use_tc_tiling_on_sc=False),
    )
    def emb_kernel(table_hbm, idx_hbm, out_hbm, idx_all, rows, gsem, osem):
        wid = lax.axis_index("s") * NUM_CORES + lax.axis_index("c")
        g0 = wid * groups_per_worker
        n = groups_per_worker
        pltpu.sync_copy(idx_hbm.at[pl.ds(g0, n)], idx_all)

        def gather_start(c, s):
            pltpu.async_copy(table_hbm.at[idx_all.at[c]], rows.at[s], gsem[s])

        def gather_wait(c, s):
            pltpu.make_async_copy(
                table_hbm.at[idx_all.at[c]], rows.at[s], gsem[s]).wait()

        def store_start(c, s):
            pltpu.async_copy(
                rows.at[pl.ds(s, 1)], out_hbm.at[pl.ds(g0 + c, 1)], osem[s])

        def store_wait(c, s):
            pltpu.make_async_copy(
                rows.at[pl.ds(s, 1)], out_hbm.at[pl.ds(g0 + c, 1)], osem[s]).wait()

        for c in range(PDIST):
            gather_start(c, c % NBUF)

        @pl.loop(0, n, step=NBUF)
        def _chunks(c0):
            for b in range(NBUF):
                c = c0 + b
                s = b
                sp = (b + PDIST) % NBUF
                cp = c + PDIST

                @pl.when(jnp.logical_and(cp < n, cp >= NBUF))
                def _():
                    store_wait(cp - NBUF, sp)

                @pl.when(cp < n)
                def _():
                    gather_start(cp, sp)

                gather_wait(c, s)

                @pl.loop(0, GROUP, unroll=4)
                def _row(r):
                    for dd in range(D_MODEL // LANES):
                        sl = pl.ds(dd * LANES, LANES)
                        rows[s, r, sl] = rows[s, r, sl] * 8.0

                store_start(c, s)

        for b in range(NBUF):
            store_wait(n - NBUF + b, b)

    return emb_kernel


def kernel(x, table):
    s0, s1 = x.shape
    total = s0 * s1
    num_groups = total // GROUP
    idx = x.reshape(total).astype(jnp.int32).reshape(num_groups, GROUP)
    emb = _build(num_groups)(table, idx)
    return emb.reshape(s0, s1, D_MODEL)
